# baseline (speedup 1.0000x reference)
import functools

import jax
import jax.numpy as jnp
from jax import lax
from jax.experimental import pallas as pl
from jax.experimental.pallas import tpu as pltpu
from jax.experimental.pallas import tpu_sc as plsc

D = 32
B = 16384
BAG = 50
NC = 2
NS = 16
NW = NC * NS
BPW = B // NW
L = 16

_mesh = plsc.VectorSubcoreMesh(core_axis_name="c", subcore_axis_name="s")


@functools.partial(
    pl.kernel,
    mesh=_mesh,
    out_type=jax.ShapeDtypeStruct((B, D), jnp.float32),
    scratch_types=[
        pltpu.VMEM((BPW * BAG,), jnp.int32),
        pltpu.VMEM((BAG, BPW), jnp.int32),
        pltpu.VMEM((BPW, D), jnp.float32),
        pltpu.SemaphoreType.DMA,
    ],
    compiler_params=pltpu.CompilerParams(
        use_tc_tiling_on_sc=False, needs_layout_passes=False
    ),
)
def _bag(idx_hbm, w_hbm, out_hbm, idx_raw, idx_t, acc, sem):
    wid = lax.axis_index("s") * NC + lax.axis_index("c")
    idx_cp = pltpu.async_copy(idx_hbm.at[wid], idx_raw, sem)
    zero = jnp.zeros((L,), jnp.float32)

    def zbody(i, carry):
        acc[i, pl.ds(0, L)] = zero
        acc[i, pl.ds(L, L)] = zero
        return carry

    lax.fori_loop(0, BPW, zbody, 0)
    idx_cp.wait()

    lane_off = lax.iota(jnp.int32, L) * BAG
    descs = []
    for j in range(BAG):
        def tbody(cc, carry, j=j):
            vals = plsc.load_gather(idx_raw, [lane_off + (cc * (L * BAG) + j)])
            idx_t[j, pl.ds(cc * L, L)] = vals
            return carry

        lax.fori_loop(0, BPW // L, tbody, 0)
        descs.append(pltpu.async_copy(w_hbm.at[idx_t.at[j]], acc, sem, add=True))
    for d in descs:
        d.wait()
    pltpu.sync_copy(acc, out_hbm.at[pl.ds(wid * BPW, BPW)])


def kernel(input, weight):
    idx = input.astype(jnp.int32).reshape(NW, BPW * BAG)
    return _bag(idx, weight)
